# baseline (speedup 1.0000x reference)
import functools
import math

import jax
import jax.numpy as jnp
from jax import lax
from jax.experimental import pallas as pl
from jax.experimental.pallas import tpu as pltpu
from jax.experimental.pallas import tpu_sc as plsc

N = 10000
E = 320000
D = 128
NP = 10240
BN = 1024
NB = NP // BN

NC = 2
NS = 16
NW = NC * NS
CH = 32
NCH = 314
EP = NW * NCH * CH
NG = CH // 16
NA = 10112
RPS = NA // NS

_INV_SQRT_D = 1.0 / math.sqrt(float(D))


def _prelu(y, a):
    return jnp.where(y > 0, y, a * y)


def _lin_body(x_ref, w_ref, b_ref, a_ref, o_ref):
    y = jnp.dot(x_ref[...], w_ref[...], preferred_element_type=jnp.float32)
    y = y + b_ref[...][None, :]
    o_ref[...] = _prelu(y, a_ref[...][None, :])


def _lin_call(x, w, b, a):
    return pl.pallas_call(
        _lin_body,
        grid=(NB,),
        in_specs=[
            pl.BlockSpec((BN, D), lambda i: (i, 0)),
            pl.BlockSpec((D, D), lambda i: (0, 0)),
            pl.BlockSpec((D,), lambda i: (0,)),
            pl.BlockSpec((D,), lambda i: (0,)),
        ],
        out_specs=pl.BlockSpec((BN, D), lambda i: (i, 0)),
        out_shape=jax.ShapeDtypeStruct((NP, D), jnp.float32),
    )(x, w, b, a)


def _qkv_body(h_ref, wq_ref, bq_ref, wkv_ref, bkv_ref, q_ref, kv_ref):
    h = h_ref[...]
    q_ref[...] = (jnp.dot(h, wq_ref[...], preferred_element_type=jnp.float32)
                  + bq_ref[...][None, :])
    kv_ref[...] = (jnp.dot(h, wkv_ref[...], preferred_element_type=jnp.float32)
                   + bkv_ref[...][None, :])


def _qkv_call(h, wq, bq, wkv, bkv):
    return pl.pallas_call(
        _qkv_body,
        grid=(NB,),
        in_specs=[
            pl.BlockSpec((BN, D), lambda i: (i, 0)),
            pl.BlockSpec((D, D), lambda i: (0, 0)),
            pl.BlockSpec((D,), lambda i: (0,)),
            pl.BlockSpec((D, 2 * D), lambda i: (0, 0)),
            pl.BlockSpec((2 * D,), lambda i: (0,)),
        ],
        out_specs=[
            pl.BlockSpec((BN, D), lambda i: (i, 0)),
            pl.BlockSpec((BN, 2 * D), lambda i: (i, 0)),
        ],
        out_shape=[
            jax.ShapeDtypeStruct((NP, D), jnp.float32),
            jax.ShapeDtypeStruct((NP, 2 * D), jnp.float32),
        ],
    )(h, wq, bq, wkv, bkv)


def _post_body(a0_ref, a1_ref, s_ref, h_ref,
               w1_ref, b1_ref, p1_ref, w2_ref, b2_ref, p2_ref, o_ref):
    s = jnp.sum(s_ref[...], axis=0)
    agg = a0_ref[0] + a1_ref[0]
    t = agg / (s[:, None] + 1e-16) + h_ref[...]
    y = jnp.dot(t, w1_ref[...], preferred_element_type=jnp.float32)
    y = _prelu(y + b1_ref[...][None, :], p1_ref[...][None, :])
    y = jnp.dot(y, w2_ref[...], preferred_element_type=jnp.float32)
    o_ref[...] = _prelu(y + b2_ref[...][None, :], p2_ref[...][None, :])


def _post_call(aggu, s_all, h, w1, b1, p1, w2, b2, p2):
    return pl.pallas_call(
        _post_body,
        grid=(NB,),
        in_specs=[
            pl.BlockSpec((1, BN, D), lambda i: (0, i, 0)),
            pl.BlockSpec((1, BN, D), lambda i: (1, i, 0)),
            pl.BlockSpec((NC, BN), lambda i: (0, i)),
            pl.BlockSpec((BN, D), lambda i: (i, 0)),
            pl.BlockSpec((D, D), lambda i: (0, 0)),
            pl.BlockSpec((D,), lambda i: (0,)),
            pl.BlockSpec((D,), lambda i: (0,)),
            pl.BlockSpec((D, D), lambda i: (0, 0)),
            pl.BlockSpec((D,), lambda i: (0,)),
            pl.BlockSpec((D,), lambda i: (0,)),
        ],
        out_specs=pl.BlockSpec((BN, D), lambda i: (i, 0)),
        out_shape=jax.ShapeDtypeStruct((NP, D), jnp.float32),
    )(aggu, aggu, s_all, h, w1, b1, p1, w2, b2, p2)


_SC_MESH = plsc.VectorSubcoreMesh(core_axis_name="c", subcore_axis_name="s")


@functools.partial(
    pl.kernel,
    mesh=_SC_MESH,
    compiler_params=pltpu.CompilerParams(needs_layout_passes=False),
    out_type=[
        jax.ShapeDtypeStruct((NC, NP), jnp.float32),
        jax.ShapeDtypeStruct((NC, NP, D), jnp.float32),
    ],
    scratch_types=[
        pltpu.VMEM((2, CH), jnp.int32),
        pltpu.VMEM((2, CH), jnp.int32),
        pltpu.VMEM((CH,), jnp.int32),
        pltpu.VMEM((CH,), jnp.int32),
        pltpu.VMEM((CH, D), jnp.float32),
        pltpu.VMEM((CH, D), jnp.float32),
        pltpu.VMEM((CH, 2 * D), jnp.float32),
        pltpu.VMEM((CH, 2 * D), jnp.float32),
        pltpu.VMEM((CH, D), jnp.float32),
        pltpu.VMEM((CH, D), jnp.float32),
        pltpu.VMEM((CH,), jnp.float32),
        pltpu.VMEM((CH,), jnp.float32),
        pltpu.VMEM_SHARED((NA,), jnp.float32),
        pltpu.VMEM_SHARED((NA, D), jnp.float32),
        pltpu.SemaphoreType.DMA,
        pltpu.SemaphoreType.DMA,
        pltpu.SemaphoreType.DMA,
        pltpu.SemaphoreType.DMA,
        pltpu.SemaphoreType.DMA,
    ],
)
def _edge_kernel(q_hbm, kv_hbm, dsx_hbm, zs_hbm, za_hbm,
                 s_out, aggu_out,
                 gix0, gix1, six0, six1,
                 qrows0, qrows1, kvrows0, kvrows1,
                 scaled0, scaled1, exc0, exc1, s_sh, aggu_sh,
                 sem_g0, sem_g1, sem_sc0, sem_sc1, sem_i):
    c = lax.axis_index("c")
    sid = lax.axis_index("s")
    wid = c * NS + sid

    gix = (gix0, gix1)
    six = (six0, six1)
    qrows = (qrows0, qrows1)
    kvrows = (kvrows0, kvrows1)
    scaled = (scaled0, scaled1)
    exc = (exc0, exc1)
    sem_g = (sem_g0, sem_g1)
    sem_sc = (sem_sc0, sem_sc1)

    pltpu.sync_copy(za_hbm.at[pl.ds(sid * RPS, RPS)],
                    aggu_sh.at[pl.ds(sid * RPS, RPS)])

    @pl.when(sid == 0)
    def _():
        pltpu.sync_copy(zs_hbm.at[pl.ds(0, NA)], s_sh)

    plsc.subcore_barrier()

    lane = lax.iota(jnp.int32, 16)
    lane_mask = [lane == l for l in range(16)]

    def issue_idx(i, b):
        pltpu.async_copy(dsx_hbm.at[wid, i], gix[b], sem_i)

    def drain_idx(i, b):
        pltpu.make_async_copy(dsx_hbm.at[wid, i], gix[b], sem_i).wait()

    def issue_qkv(b):
        pltpu.async_copy(q_hbm.at[gix[b].at[0]], qrows[b], sem_g[b])
        pltpu.async_copy(kv_hbm.at[gix[b].at[1]], kvrows[b], sem_g[b])

    def drain_qkv(b):
        pltpu.make_async_copy(q_hbm.at[gix[b].at[0]], qrows[b],
                              sem_g[b]).wait()
        pltpu.make_async_copy(kv_hbm.at[gix[b].at[1]], kvrows[b],
                              sem_g[b]).wait()

    def issue_sc(b):
        pltpu.async_copy(exc[b], s_sh.at[six[b]], sem_sc[b], add=True)
        pltpu.async_copy(scaled[b], aggu_sh.at[six[b]], sem_sc[b], add=True)

    def drain_sc(b):
        pltpu.make_async_copy(exc[b], s_sh.at[six[b]], sem_sc[b]).wait()
        pltpu.make_async_copy(scaled[b], aggu_sh.at[six[b]], sem_sc[b]).wait()

    def chunk_work(i, b):
        drain_qkv(b)

        @pl.when(i + 1 < NCH)
        def _():
            drain_idx(i + 1, 1 - b)
            issue_qkv(1 - b)

        qr, kvr = qrows[b], kvrows[b]
        tots = []
        for e in range(CH):
            acc0 = qr[e, pl.ds(0, 16)] * kvr[e, pl.ds(0, 16)]
            acc1 = qr[e, pl.ds(16, 16)] * kvr[e, pl.ds(16, 16)]
            for j in range(2, D // 16, 2):
                acc0 = acc0 + qr[e, pl.ds(16 * j, 16)] * kvr[e, pl.ds(16 * j, 16)]
                acc1 = acc1 + (qr[e, pl.ds(16 * (j + 1), 16)]
                               * kvr[e, pl.ds(16 * (j + 1), 16)])
            tots.append(jnp.sum(acc0 + acc1))
        exs = []
        for g in range(NG):
            alpha = jnp.zeros((16,), jnp.float32)
            for l in range(16):
                alpha = jnp.where(lane_mask[l], tots[16 * g + l], alpha)
            exs.append(jnp.exp(alpha * _INV_SQRT_D))

        @pl.when(i >= 2)
        def _():
            drain_sc(b)

        for g in range(NG):
            dstv = gix[b][0, pl.ds(16 * g, 16)]
            six[b][pl.ds(16 * g, 16)] = dstv
            exc[b][pl.ds(16 * g, 16)] = exs[g]

        @pl.when(i + 2 < NCH)
        def _():
            issue_idx(i + 2, b)

        scb = scaled[b]
        for e in range(CH):
            exv = jnp.full((16,), exs[e // 16][e % 16])
            for j in range(D // 16):
                scb[e, pl.ds(16 * j, 16)] = (
                    kvr[e, pl.ds(D + 16 * j, 16)] * exv)

        issue_sc(b)

    pltpu.sync_copy(dsx_hbm.at[wid, 0], gix0)
    issue_idx(1, 1)
    issue_qkv(0)

    def pair_body(p, carry):
        chunk_work(2 * p, 0)
        chunk_work(2 * p + 1, 1)
        return carry

    lax.fori_loop(0, NCH // 2, pair_body, 0)

    drain_sc(0)
    drain_sc(1)

    plsc.subcore_barrier()

    pltpu.sync_copy(aggu_sh.at[pl.ds(sid * RPS, RPS)],
                    aggu_out.at[c, pl.ds(sid * RPS, RPS)])

    @pl.when(sid == 0)
    def _():
        pltpu.sync_copy(s_sh, s_out.at[c, pl.ds(0, NA)])


def kernel(x, edge_index, params):
    pad = EP - E
    dst = jnp.concatenate(
        [edge_index[1], jnp.full((pad,), N, jnp.int32)]).reshape(NW, NCH, CH)
    src = jnp.concatenate(
        [edge_index[0], jnp.zeros((pad,), jnp.int32)]).reshape(NW, NCH, CH)
    dsx = jnp.stack([dst, src], axis=2)

    xp = jnp.zeros((NP, D), jnp.float32).at[:N].set(x)
    zs = jnp.zeros((NP,), jnp.float32)
    za = jnp.zeros((NP, D), jnp.float32)

    h = xp
    for lp in params['lin']:
        h = _lin_call(h, lp['W'], lp['b'], lp['a'])

    for rp in params['rgit']:
        wkv = jnp.concatenate([rp['Wk'], rp['Wv']], axis=1)
        bkv = jnp.concatenate([rp['bk'], rp['bv']])
        q, kv = _qkv_call(h, rp['Wq'], rp['bq'], wkv, bkv)
        s_parts, aggu_parts = _edge_kernel(q, kv, dsx, zs, za)
        h = _post_call(aggu_parts, s_parts, h,
                       rp['nn_W1'], rp['nn_b1'], rp['nn_a1'],
                       rp['nn_W2'], rp['nn_b2'], rp['nn_a2'])

    return h[:N]


# speedup vs baseline: 1.0437x; 1.0437x over previous
import functools
import math

import jax
import jax.numpy as jnp
from jax import lax
from jax.experimental import pallas as pl
from jax.experimental.pallas import tpu as pltpu
from jax.experimental.pallas import tpu_sc as plsc

N = 10000
E = 320000
D = 128
NP = 10240
BN = 1024
NB = NP // BN

NC = 2
NS = 16
NW = NC * NS
CH = 32
NCH = 314
EP = NW * NCH * CH
NG = CH // 16
NA = 10112
RPS = NA // NS

_INV_SQRT_D = 1.0 / math.sqrt(float(D))


def _prelu(y, a):
    return jnp.where(y > 0, y, a * y)


def _lin_body(x_ref, w_ref, b_ref, a_ref, o_ref):
    y = jnp.dot(x_ref[...], w_ref[...], preferred_element_type=jnp.float32)
    y = y + b_ref[...][None, :]
    o_ref[...] = _prelu(y, a_ref[...][None, :])


def _lin_call(x, w, b, a):
    return pl.pallas_call(
        _lin_body,
        grid=(NB,),
        in_specs=[
            pl.BlockSpec((BN, D), lambda i: (i, 0)),
            pl.BlockSpec((D, D), lambda i: (0, 0)),
            pl.BlockSpec((D,), lambda i: (0,)),
            pl.BlockSpec((D,), lambda i: (0,)),
        ],
        out_specs=pl.BlockSpec((BN, D), lambda i: (i, 0)),
        out_shape=jax.ShapeDtypeStruct((NP, D), jnp.float32),
    )(x, w, b, a)


def _qkv_body(h_ref, wq_ref, bq_ref, wk_ref, bk_ref, wv_ref, bv_ref,
              q_ref, k_ref, v_ref):
    h = h_ref[...]
    q_ref[...] = (jnp.dot(h, wq_ref[...], preferred_element_type=jnp.float32)
                  + bq_ref[...][None, :])
    k_ref[...] = (jnp.dot(h, wk_ref[...], preferred_element_type=jnp.float32)
                  + bk_ref[...][None, :])
    v_ref[...] = (jnp.dot(h, wv_ref[...], preferred_element_type=jnp.float32)
                  + bv_ref[...][None, :])


def _qkv_call(h, wq, bq, wk, bk, wv, bv):
    mat = pl.BlockSpec((D, D), lambda i: (0, 0))
    vec = pl.BlockSpec((D,), lambda i: (0,))
    blk = pl.BlockSpec((BN, D), lambda i: (i, 0))
    return pl.pallas_call(
        _qkv_body,
        grid=(NB,),
        in_specs=[blk, mat, vec, mat, vec, mat, vec],
        out_specs=[blk, blk, blk],
        out_shape=[jax.ShapeDtypeStruct((NP, D), jnp.float32)] * 3,
    )(h, wq, bq, wk, bk, wv, bv)


def _post_body(a0_ref, a1_ref, s_ref, h_ref,
               w1_ref, b1_ref, p1_ref, w2_ref, b2_ref, p2_ref, o_ref):
    s = jnp.sum(s_ref[...], axis=0)
    agg = a0_ref[0] + a1_ref[0]
    t = agg / (s[:, None] + 1e-16) + h_ref[...]
    y = jnp.dot(t, w1_ref[...], preferred_element_type=jnp.float32)
    y = _prelu(y + b1_ref[...][None, :], p1_ref[...][None, :])
    y = jnp.dot(y, w2_ref[...], preferred_element_type=jnp.float32)
    o_ref[...] = _prelu(y + b2_ref[...][None, :], p2_ref[...][None, :])


def _post_call(aggu, s_all, h, w1, b1, p1, w2, b2, p2):
    return pl.pallas_call(
        _post_body,
        grid=(NB,),
        in_specs=[
            pl.BlockSpec((1, BN, D), lambda i: (0, i, 0)),
            pl.BlockSpec((1, BN, D), lambda i: (1, i, 0)),
            pl.BlockSpec((NC, BN), lambda i: (0, i)),
            pl.BlockSpec((BN, D), lambda i: (i, 0)),
            pl.BlockSpec((D, D), lambda i: (0, 0)),
            pl.BlockSpec((D,), lambda i: (0,)),
            pl.BlockSpec((D,), lambda i: (0,)),
            pl.BlockSpec((D, D), lambda i: (0, 0)),
            pl.BlockSpec((D,), lambda i: (0,)),
            pl.BlockSpec((D,), lambda i: (0,)),
        ],
        out_specs=pl.BlockSpec((BN, D), lambda i: (i, 0)),
        out_shape=jax.ShapeDtypeStruct((NP, D), jnp.float32),
    )(aggu, aggu, s_all, h, w1, b1, p1, w2, b2, p2)


_SC_MESH = plsc.VectorSubcoreMesh(core_axis_name="c", subcore_axis_name="s")


@functools.partial(
    pl.kernel,
    mesh=_SC_MESH,
    compiler_params=pltpu.CompilerParams(needs_layout_passes=False),
    out_type=[
        jax.ShapeDtypeStruct((NC, NP), jnp.float32),
        jax.ShapeDtypeStruct((NC, NP, D), jnp.float32),
    ],
    scratch_types=[
        pltpu.VMEM((2, CH), jnp.int32),
        pltpu.VMEM((2, CH), jnp.int32),
        pltpu.VMEM((CH,), jnp.int32),
        pltpu.VMEM((CH,), jnp.int32),
        pltpu.VMEM((CH, D), jnp.float32),
        pltpu.VMEM((CH, D), jnp.float32),
        pltpu.VMEM((CH, D), jnp.float32),
        pltpu.VMEM((CH, D), jnp.float32),
        pltpu.VMEM((CH, D), jnp.float32),
        pltpu.VMEM((CH, D), jnp.float32),
        pltpu.VMEM((CH, D), jnp.float32),
        pltpu.VMEM((CH, D), jnp.float32),
        pltpu.VMEM((CH,), jnp.float32),
        pltpu.VMEM((CH,), jnp.float32),
        pltpu.VMEM_SHARED((NA,), jnp.float32),
        pltpu.VMEM_SHARED((NA, D), jnp.float32),
        pltpu.SemaphoreType.DMA,
        pltpu.SemaphoreType.DMA,
        pltpu.SemaphoreType.DMA,
        pltpu.SemaphoreType.DMA,
        pltpu.SemaphoreType.DMA,
    ],
)
def _edge_kernel(q_hbm, k_hbm, v_hbm, dsx_hbm, zs_hbm, za_hbm,
                 s_out, aggu_out,
                 gix0, gix1, six0, six1,
                 qrows0, qrows1, krows0, krows1, vrows0, vrows1,
                 scaled0, scaled1, exc0, exc1, s_sh, aggu_sh,
                 sem_g0, sem_g1, sem_sc0, sem_sc1, sem_i):
    c = lax.axis_index("c")
    sid = lax.axis_index("s")
    wid = c * NS + sid

    gix = (gix0, gix1)
    six = (six0, six1)
    qrows = (qrows0, qrows1)
    krows = (krows0, krows1)
    vrows = (vrows0, vrows1)
    scaled = (scaled0, scaled1)
    exc = (exc0, exc1)
    sem_g = (sem_g0, sem_g1)
    sem_sc = (sem_sc0, sem_sc1)

    pltpu.sync_copy(za_hbm.at[pl.ds(sid * RPS, RPS)],
                    aggu_sh.at[pl.ds(sid * RPS, RPS)])

    @pl.when(sid == 0)
    def _():
        pltpu.sync_copy(zs_hbm.at[pl.ds(0, NA)], s_sh)

    plsc.subcore_barrier()

    lane = lax.iota(jnp.int32, 16)
    lane_mask = [lane == l for l in range(16)]

    def issue_idx(i, b):
        pltpu.async_copy(dsx_hbm.at[wid, i], gix[b], sem_i)

    def drain_idx(i, b):
        pltpu.make_async_copy(dsx_hbm.at[wid, i], gix[b], sem_i).wait()

    def issue_qkv(b):
        pltpu.async_copy(q_hbm.at[gix[b].at[0]], qrows[b], sem_g[b])
        pltpu.async_copy(k_hbm.at[gix[b].at[1]], krows[b], sem_g[b])
        pltpu.async_copy(v_hbm.at[gix[b].at[1]], vrows[b], sem_g[b])

    def drain_qkv(b):
        pltpu.make_async_copy(q_hbm.at[gix[b].at[0]], qrows[b],
                              sem_g[b]).wait()
        pltpu.make_async_copy(k_hbm.at[gix[b].at[1]], krows[b],
                              sem_g[b]).wait()
        pltpu.make_async_copy(v_hbm.at[gix[b].at[1]], vrows[b],
                              sem_g[b]).wait()

    def issue_sc(b):
        pltpu.async_copy(exc[b], s_sh.at[six[b]], sem_sc[b], add=True)
        pltpu.async_copy(scaled[b], aggu_sh.at[six[b]], sem_sc[b], add=True)

    def drain_sc(b):
        pltpu.make_async_copy(exc[b], s_sh.at[six[b]], sem_sc[b]).wait()
        pltpu.make_async_copy(scaled[b], aggu_sh.at[six[b]], sem_sc[b]).wait()

    def chunk_work(i, b):
        drain_qkv(b)

        @pl.when(i + 1 < NCH)
        def _():
            drain_idx(i + 1, 1 - b)
            issue_qkv(1 - b)

        qr, kr, vr = qrows[b], krows[b], vrows[b]
        tots = []
        for e in range(CH):
            acc0 = qr[e, pl.ds(0, 16)] * kr[e, pl.ds(0, 16)]
            acc1 = qr[e, pl.ds(16, 16)] * kr[e, pl.ds(16, 16)]
            for j in range(2, D // 16, 2):
                acc0 = acc0 + qr[e, pl.ds(16 * j, 16)] * kr[e, pl.ds(16 * j, 16)]
                acc1 = acc1 + (qr[e, pl.ds(16 * (j + 1), 16)]
                               * kr[e, pl.ds(16 * (j + 1), 16)])
            tots.append(jnp.sum(acc0 + acc1))
        exs = []
        for g in range(NG):
            alpha = jnp.zeros((16,), jnp.float32)
            for l in range(16):
                alpha = jnp.where(lane_mask[l], tots[16 * g + l], alpha)
            exs.append(jnp.exp(alpha * _INV_SQRT_D))

        @pl.when(i >= 2)
        def _():
            drain_sc(b)

        for g in range(NG):
            dstv = gix[b][0, pl.ds(16 * g, 16)]
            six[b][pl.ds(16 * g, 16)] = dstv
            exc[b][pl.ds(16 * g, 16)] = exs[g]

        @pl.when(i + 2 < NCH)
        def _():
            issue_idx(i + 2, b)

        scb = scaled[b]
        for e in range(CH):
            exv = jnp.full((16,), exs[e // 16][e % 16])
            for j in range(D // 16):
                scb[e, pl.ds(16 * j, 16)] = vr[e, pl.ds(16 * j, 16)] * exv

        issue_sc(b)

    pltpu.sync_copy(dsx_hbm.at[wid, 0], gix0)
    issue_idx(1, 1)
    issue_qkv(0)

    def pair_body(p, carry):
        chunk_work(2 * p, 0)
        chunk_work(2 * p + 1, 1)
        return carry

    lax.fori_loop(0, NCH // 2, pair_body, 0)

    drain_sc(0)
    drain_sc(1)

    plsc.subcore_barrier()

    pltpu.sync_copy(aggu_sh.at[pl.ds(sid * RPS, RPS)],
                    aggu_out.at[c, pl.ds(sid * RPS, RPS)])

    @pl.when(sid == 0)
    def _():
        pltpu.sync_copy(s_sh, s_out.at[c, pl.ds(0, NA)])


def kernel(x, edge_index, params):
    pad = EP - E
    dst = jnp.concatenate(
        [edge_index[1], jnp.full((pad,), N, jnp.int32)]).reshape(NW, NCH, CH)
    src = jnp.concatenate(
        [edge_index[0], jnp.zeros((pad,), jnp.int32)]).reshape(NW, NCH, CH)
    dsx = jnp.stack([dst, src], axis=2)

    xp = jnp.zeros((NP, D), jnp.float32).at[:N].set(x)
    zs = jnp.zeros((NP,), jnp.float32)
    za = jnp.zeros((NP, D), jnp.float32)

    h = xp
    for lp in params['lin']:
        h = _lin_call(h, lp['W'], lp['b'], lp['a'])

    for rp in params['rgit']:
        q, k, v = _qkv_call(h, rp['Wq'], rp['bq'], rp['Wk'], rp['bk'],
                            rp['Wv'], rp['bv'])
        s_parts, aggu_parts = _edge_kernel(q, k, v, dsx, zs, za)
        h = _post_call(aggu_parts, s_parts, h,
                       rp['nn_W1'], rp['nn_b1'], rp['nn_a1'],
                       rp['nn_W2'], rp['nn_b2'], rp['nn_a2'])

    return h[:N]


# speedup vs baseline: 1.1007x; 1.0546x over previous
import functools
import math

import jax
import jax.numpy as jnp
from jax import lax
from jax.experimental import pallas as pl
from jax.experimental.pallas import tpu as pltpu
from jax.experimental.pallas import tpu_sc as plsc

N = 10000
E = 320000
D = 128
NP = 10240
BN = 1024
NB = NP // BN

NC = 2
NS = 16
NW = NC * NS
CH = 32
NCH = 314
EP = NW * NCH * CH
NG = CH // 16
NA = 10112
RPS = NA // NS

_INV_SQRT_D = 1.0 / math.sqrt(float(D))


def _prelu(y, a):
    return jnp.where(y > 0, y, a * y)


def _lin_body(x_ref, w_ref, b_ref, a_ref, o_ref):
    y = jnp.dot(x_ref[...], w_ref[...], preferred_element_type=jnp.float32)
    y = y + b_ref[...][None, :]
    o_ref[...] = _prelu(y, a_ref[...][None, :])


def _lin_call(x, w, b, a):
    return pl.pallas_call(
        _lin_body,
        grid=(NB,),
        in_specs=[
            pl.BlockSpec((BN, D), lambda i: (i, 0)),
            pl.BlockSpec((D, D), lambda i: (0, 0)),
            pl.BlockSpec((D,), lambda i: (0,)),
            pl.BlockSpec((D,), lambda i: (0,)),
        ],
        out_specs=pl.BlockSpec((BN, D), lambda i: (i, 0)),
        out_shape=jax.ShapeDtypeStruct((NP, D), jnp.float32),
    )(x, w, b, a)


def _qkv_body(h_ref, wq_ref, bq_ref, wk_ref, bk_ref, wv_ref, bv_ref,
              q_ref, k_ref, v_ref):
    h = h_ref[...]
    q_ref[...] = (jnp.dot(h, wq_ref[...], preferred_element_type=jnp.float32)
                  + bq_ref[...][None, :])
    k_ref[...] = (jnp.dot(h, wk_ref[...], preferred_element_type=jnp.float32)
                  + bk_ref[...][None, :])
    v_ref[...] = (jnp.dot(h, wv_ref[...], preferred_element_type=jnp.float32)
                  + bv_ref[...][None, :])


def _qkv_call(h, wq, bq, wk, bk, wv, bv):
    mat = pl.BlockSpec((D, D), lambda i: (0, 0))
    vec = pl.BlockSpec((D,), lambda i: (0,))
    blk = pl.BlockSpec((BN, D), lambda i: (i, 0))
    return pl.pallas_call(
        _qkv_body,
        grid=(NB,),
        in_specs=[blk, mat, vec, mat, vec, mat, vec],
        out_specs=[blk, blk, blk],
        out_shape=[jax.ShapeDtypeStruct((NP, D), jnp.float32)] * 3,
    )(h, wq, bq, wk, bk, wv, bv)


def _post_body(a0_ref, a1_ref, s_ref, h_ref,
               w1_ref, b1_ref, p1_ref, w2_ref, b2_ref, p2_ref, o_ref):
    s = jnp.sum(s_ref[...], axis=0)
    agg = a0_ref[0] + a1_ref[0]
    t = agg / (s[:, None] + 1e-16) + h_ref[...]
    y = jnp.dot(t, w1_ref[...], preferred_element_type=jnp.float32)
    y = _prelu(y + b1_ref[...][None, :], p1_ref[...][None, :])
    y = jnp.dot(y, w2_ref[...], preferred_element_type=jnp.float32)
    o_ref[...] = _prelu(y + b2_ref[...][None, :], p2_ref[...][None, :])


def _post_call(aggu, s_all, h, w1, b1, p1, w2, b2, p2):
    return pl.pallas_call(
        _post_body,
        grid=(NB,),
        in_specs=[
            pl.BlockSpec((1, BN, D), lambda i: (0, i, 0)),
            pl.BlockSpec((1, BN, D), lambda i: (1, i, 0)),
            pl.BlockSpec((NC, BN), lambda i: (0, i)),
            pl.BlockSpec((BN, D), lambda i: (i, 0)),
            pl.BlockSpec((D, D), lambda i: (0, 0)),
            pl.BlockSpec((D,), lambda i: (0,)),
            pl.BlockSpec((D,), lambda i: (0,)),
            pl.BlockSpec((D, D), lambda i: (0, 0)),
            pl.BlockSpec((D,), lambda i: (0,)),
            pl.BlockSpec((D,), lambda i: (0,)),
        ],
        out_specs=pl.BlockSpec((BN, D), lambda i: (i, 0)),
        out_shape=jax.ShapeDtypeStruct((NP, D), jnp.float32),
    )(aggu, aggu, s_all, h, w1, b1, p1, w2, b2, p2)


_SC_MESH = plsc.VectorSubcoreMesh(core_axis_name="c", subcore_axis_name="s")


@functools.partial(
    pl.kernel,
    mesh=_SC_MESH,
    compiler_params=pltpu.CompilerParams(needs_layout_passes=False),
    out_type=[
        jax.ShapeDtypeStruct((NC, NP), jnp.float32),
        jax.ShapeDtypeStruct((NC, NP, D), jnp.float32),
    ],
    scratch_types=[
        pltpu.VMEM((2, CH), jnp.int32),
        pltpu.VMEM((2, CH), jnp.int32),
        pltpu.VMEM((CH,), jnp.int32),
        pltpu.VMEM((CH,), jnp.int32),
        pltpu.VMEM((CH, D), jnp.float32),
        pltpu.VMEM((CH, D), jnp.float32),
        pltpu.VMEM((CH, D), jnp.float32),
        pltpu.VMEM((CH, D), jnp.float32),
        pltpu.VMEM((CH, D), jnp.float32),
        pltpu.VMEM((CH, D), jnp.float32),
        pltpu.VMEM((CH, D), jnp.float32),
        pltpu.VMEM((CH,), jnp.float32),
        pltpu.VMEM_SHARED((NA,), jnp.float32),
        pltpu.VMEM_SHARED((NA, D), jnp.float32),
        pltpu.SemaphoreType.DMA,
        pltpu.SemaphoreType.DMA,
        pltpu.SemaphoreType.DMA,
        pltpu.SemaphoreType.DMA,
    ],
)
def _edge_kernel(q_hbm, k_hbm, v_hbm, dsx_hbm, zs_hbm, za_hbm,
                 s_out, aggu_out,
                 gix0, gix1, six0, six1,
                 qrows0, qrows1, krows0, krows1, vrows0, vrows1,
                 scaled, exc, s_sh, aggu_sh,
                 sem_g0, sem_g1, sem_sc, sem_i):
    c = lax.axis_index("c")
    sid = lax.axis_index("s")
    wid = c * NS + sid

    gix = (gix0, gix1)
    six = (six0, six1)
    qrows = (qrows0, qrows1)
    krows = (krows0, krows1)
    vrows = (vrows0, vrows1)
    sem_g = (sem_g0, sem_g1)

    pltpu.sync_copy(za_hbm.at[pl.ds(sid * RPS, RPS)],
                    aggu_sh.at[pl.ds(sid * RPS, RPS)])

    @pl.when(sid == 0)
    def _():
        pltpu.sync_copy(zs_hbm.at[pl.ds(0, NA)], s_sh)

    plsc.subcore_barrier()

    lane = lax.iota(jnp.int32, 16)
    lane_mask = [lane == l for l in range(16)]

    def issue_idx(i, b):
        pltpu.async_copy(dsx_hbm.at[wid, i], gix[b], sem_i)

    def drain_idx(i, b):
        pltpu.make_async_copy(dsx_hbm.at[wid, i], gix[b], sem_i).wait()

    def issue_qkv(b):
        pltpu.async_copy(q_hbm.at[gix[b].at[0]], qrows[b], sem_g[b])
        pltpu.async_copy(k_hbm.at[gix[b].at[1]], krows[b], sem_g[b])
        pltpu.async_copy(v_hbm.at[gix[b].at[1]], vrows[b], sem_g[b])

    def drain_qkv(b):
        pltpu.make_async_copy(q_hbm.at[gix[b].at[0]], qrows[b],
                              sem_g[b]).wait()
        pltpu.make_async_copy(k_hbm.at[gix[b].at[1]], krows[b],
                              sem_g[b]).wait()
        pltpu.make_async_copy(v_hbm.at[gix[b].at[1]], vrows[b],
                              sem_g[b]).wait()

    def issue_sc(b):
        pltpu.async_copy(exc, s_sh.at[six[b]], sem_sc, add=True)
        pltpu.async_copy(scaled, aggu_sh.at[six[b]], sem_sc, add=True)

    def drain_sc(b):
        pltpu.make_async_copy(exc, s_sh.at[six[b]], sem_sc).wait()
        pltpu.make_async_copy(scaled, aggu_sh.at[six[b]], sem_sc).wait()

    def chunk_work(i, b):
        drain_qkv(b)

        @pl.when(i + 1 < NCH)
        def _():
            drain_idx(i + 1, 1 - b)
            issue_qkv(1 - b)

        qr, kr, vr = qrows[b], krows[b], vrows[b]
        tots = []
        for e in range(CH):
            acc0 = qr[e, pl.ds(0, 16)] * kr[e, pl.ds(0, 16)]
            acc1 = qr[e, pl.ds(16, 16)] * kr[e, pl.ds(16, 16)]
            for j in range(2, D // 16, 2):
                acc0 = acc0 + qr[e, pl.ds(16 * j, 16)] * kr[e, pl.ds(16 * j, 16)]
                acc1 = acc1 + (qr[e, pl.ds(16 * (j + 1), 16)]
                               * kr[e, pl.ds(16 * (j + 1), 16)])
            tots.append(jnp.sum(acc0 + acc1))
        exs = []
        for g in range(NG):
            alpha = jnp.zeros((16,), jnp.float32)
            for l in range(16):
                alpha = jnp.where(lane_mask[l], tots[16 * g + l], alpha)
            exs.append(jnp.exp(alpha * _INV_SQRT_D))

        @pl.when(i >= 1)
        def _():
            drain_sc(1 - b)

        for g in range(NG):
            dstv = gix[b][0, pl.ds(16 * g, 16)]
            six[b][pl.ds(16 * g, 16)] = dstv
            exc[pl.ds(16 * g, 16)] = exs[g]

        @pl.when(i + 2 < NCH)
        def _():
            issue_idx(i + 2, b)

        for e in range(CH):
            exv = jnp.full((16,), exs[e // 16][e % 16])
            for j in range(D // 16):
                scaled[e, pl.ds(16 * j, 16)] = vr[e, pl.ds(16 * j, 16)] * exv

        issue_sc(b)

    pltpu.sync_copy(dsx_hbm.at[wid, 0], gix0)
    issue_idx(1, 1)
    issue_qkv(0)

    def pair_body(p, carry):
        chunk_work(2 * p, 0)
        chunk_work(2 * p + 1, 1)
        return carry

    lax.fori_loop(0, NCH // 2, pair_body, 0)

    drain_sc(1)

    plsc.subcore_barrier()

    pltpu.sync_copy(aggu_sh.at[pl.ds(sid * RPS, RPS)],
                    aggu_out.at[c, pl.ds(sid * RPS, RPS)])

    @pl.when(sid == 0)
    def _():
        pltpu.sync_copy(s_sh, s_out.at[c, pl.ds(0, NA)])


def kernel(x, edge_index, params):
    pad = EP - E
    dst = jnp.concatenate(
        [edge_index[1], jnp.full((pad,), N, jnp.int32)]).reshape(NW, NCH, CH)
    src = jnp.concatenate(
        [edge_index[0], jnp.zeros((pad,), jnp.int32)]).reshape(NW, NCH, CH)
    dsx = jnp.stack([dst, src], axis=2)

    xp = jnp.zeros((NP, D), jnp.float32).at[:N].set(x)
    zs = jnp.zeros((NP,), jnp.float32)
    za = jnp.zeros((NP, D), jnp.float32)

    h = xp
    for lp in params['lin']:
        h = _lin_call(h, lp['W'], lp['b'], lp['a'])

    for rp in params['rgit']:
        q, k, v = _qkv_call(h, rp['Wq'], rp['bq'], rp['Wk'], rp['bk'],
                            rp['Wv'], rp['bv'])
        s_parts, aggu_parts = _edge_kernel(q, k, v, dsx, zs, za)
        h = _post_call(aggu_parts, s_parts, h,
                       rp['nn_W1'], rp['nn_b1'], rp['nn_a1'],
                       rp['nn_W2'], rp['nn_b2'], rp['nn_a2'])

    return h[:N]
